# baseline (speedup 1.0000x reference)
import functools

import jax
import jax.numpy as jnp
from jax import lax
from jax.experimental import pallas as pl
from jax.experimental.pallas import tpu as pltpu
from jax.experimental.pallas import tpu_sc as plsc

N_NODES = 10000
IN_F = 128
HID = 128
OUT = 64
N_EDGES = 320000

_NC = 2
_NS = 16
_NW = _NC * _NS
_B = 128
_NB = 80
_EPT = _NB * _B
_E_PAD = _NW * _EPT
_NBUF = 4
_D = 64
_RPT = 632
_RPT_LAST = N_NODES - _RPT * (_NS - 1)

_mesh = plsc.VectorSubcoreMesh(core_axis_name="c", subcore_axis_name="s")


@functools.partial(
    pl.kernel,
    mesh=_mesh,
    compiler_params=pltpu.CompilerParams(use_tc_tiling_on_sc=False),
    out_type=jax.ShapeDtypeStruct((_NC, N_NODES, _D), jnp.float32),
    scratch_types=[
        pltpu.VMEM((_NB, _B), jnp.int32),
        pltpu.VMEM((_NB, _B), jnp.int32),
        pltpu.VMEM((_NB, _B), jnp.float32),
        pltpu.VMEM((_NBUF, _B, _D), jnp.float32),
        pltpu.VMEM_SHARED((N_NODES, _D), jnp.float32),
        pltpu.SemaphoreType.DMA,
        pltpu.SemaphoreType.DMA,
    ],
)
def _spmm(sup_hbm, src_hbm, dst_hbm, w_hbm, zero_hbm, out_hbm,
          src_v, dst_v, w_v, rows_v, acc, gsem, ssem):
    c = lax.axis_index("c")
    s = lax.axis_index("s")
    wid = s * _NC + c

    pltpu.sync_copy(src_hbm.at[wid], src_v)
    pltpu.sync_copy(dst_hbm.at[wid], dst_v)
    pltpu.sync_copy(w_hbm.at[wid], w_v)

    rbase = pl.multiple_of(s * _RPT, 8)

    @pl.when(s < _NS - 1)
    def _():
        pltpu.sync_copy(zero_hbm.at[pl.ds(rbase, _RPT)],
                        acc.at[pl.ds(rbase, _RPT)])

    @pl.when(s == _NS - 1)
    def _():
        pltpu.sync_copy(zero_hbm.at[pl.ds(rbase, _RPT_LAST)],
                        acc.at[pl.ds(rbase, _RPT_LAST)])

    plsc.subcore_barrier()

    def gather(b, u):
        return pltpu.async_copy(sup_hbm.at[src_v.at[b]], rows_v.at[u], gsem)

    for u in range(_NBUF):
        gather(u, u)

    def block(t, carry):
        for u in range(_NBUF):
            b = t * _NBUF + u
            pltpu.make_async_copy(sup_hbm.at[src_v.at[b]], rows_v.at[u],
                                  gsem).wait()

            def scale(g, cc):
                wvec = w_v[b, pl.ds(g * 16, 16)]
                for j in range(16):
                    wspl = lax.broadcast(wvec[j], (16,))
                    e = g * 16 + j
                    for k in range(_D // 16):
                        sl = pl.ds(k * 16, 16)
                        rows_v[u, e, sl] = rows_v[u, e, sl] * wspl
                return cc

            lax.fori_loop(0, _B // 16, scale, 0)
            pltpu.async_copy(rows_v.at[u], acc.at[dst_v.at[b]], ssem,
                             add=True)

        for u in range(_NBUF):
            b = t * _NBUF + u
            pltpu.make_async_copy(rows_v.at[u], acc.at[dst_v.at[b]],
                                  ssem).wait()

            @pl.when(t < _NB // _NBUF - 1)
            def _():
                gather(b + _NBUF, u)

        return carry

    lax.fori_loop(0, _NB // _NBUF, block, 0)
    plsc.subcore_barrier()

    @pl.when(s < _NS - 1)
    def _():
        pltpu.sync_copy(acc.at[pl.ds(rbase, _RPT)],
                        out_hbm.at[c, pl.ds(rbase, _RPT)])

    @pl.when(s == _NS - 1)
    def _():
        pltpu.sync_copy(acc.at[pl.ds(rbase, _RPT_LAST)],
                        out_hbm.at[c, pl.ds(rbase, _RPT_LAST)])


def _mm_split_body(x_ref, w_ref, oa_ref, ob_ref):
    sup = jnp.dot(x_ref[...], w_ref[...], preferred_element_type=jnp.float32)
    oa_ref[...] = sup[:, :_D]
    ob_ref[...] = sup[:, _D:]


def _sum_relu_mm_body(pa_ref, pb_ref, w_ref, o_ref):
    h = jnp.concatenate(
        [jnp.maximum(pa_ref[0] + pa_ref[1], 0.0),
         jnp.maximum(pb_ref[0] + pb_ref[1], 0.0)], axis=-1)
    o_ref[...] = jnp.dot(h, w_ref[...], preferred_element_type=jnp.float32)


def _sum_relu_softmax_body(p_ref, o_ref):
    z = jnp.maximum(p_ref[0] + p_ref[1], 0.0)
    z = z - jnp.max(z, axis=-1, keepdims=True)
    ez = jnp.exp(z)
    o_ref[...] = ez / jnp.sum(ez, axis=-1, keepdims=True)


def kernel(x, edge_index, edge_weight, W1, W2):
    src = edge_index[0].astype(jnp.int32)
    dst = edge_index[1].astype(jnp.int32)
    w = edge_weight.astype(jnp.float32)

    pad = _E_PAD - N_EDGES
    src_p = jnp.concatenate(
        [src, jnp.zeros((pad,), jnp.int32)]).reshape(_NW, _NB, _B)
    dst_p = jnp.concatenate(
        [dst, jnp.arange(pad, dtype=jnp.int32) % N_NODES]
    ).reshape(_NW, _NB, _B)
    w_p = jnp.concatenate(
        [w, jnp.zeros((pad,), jnp.float32)]).reshape(_NW, _NB, _B)

    zeros64 = jnp.zeros((N_NODES, _D), jnp.float32)

    s1a, s1b = pl.pallas_call(
        _mm_split_body,
        out_shape=(jax.ShapeDtypeStruct((N_NODES, _D), jnp.float32),
                   jax.ShapeDtypeStruct((N_NODES, _D), jnp.float32)),
    )(x, W1)

    p1a = _spmm(s1a, src_p, dst_p, w_p, zeros64)
    p1b = _spmm(s1b, src_p, dst_p, w_p, zeros64)

    support2 = pl.pallas_call(
        _sum_relu_mm_body,
        out_shape=jax.ShapeDtypeStruct((N_NODES, OUT), jnp.float32),
    )(p1a, p1b, W2)

    p2 = _spmm(support2, src_p, dst_p, w_p, zeros64)

    return pl.pallas_call(
        _sum_relu_softmax_body,
        out_shape=jax.ShapeDtypeStruct((N_NODES, OUT), jnp.float32),
    )(p2)


# speedup vs baseline: 1.7383x; 1.7383x over previous
import functools

import jax
import jax.numpy as jnp
from jax import lax
from jax.experimental import pallas as pl
from jax.experimental.pallas import tpu as pltpu
from jax.experimental.pallas import tpu_sc as plsc

N_NODES = 10000
IN_F = 128
HID = 128
OUT = 64
N_EDGES = 320000

_NC = 2
_NS = 16
_NW = _NC * _NS
_B = 128
_NB = 80
_EPT = _NB * _B
_E_PAD = _NW * _EPT
_NBUF = 4
_D = 64
_RPT = 632
_RPT_LAST = N_NODES - _RPT * (_NS - 1)

_mesh = plsc.VectorSubcoreMesh(core_axis_name="c", subcore_axis_name="s")


@functools.partial(
    pl.kernel,
    mesh=_mesh,
    compiler_params=pltpu.CompilerParams(use_tc_tiling_on_sc=False),
    out_type=jax.ShapeDtypeStruct((_NC, N_NODES, _D), jnp.float32),
    scratch_types=[
        pltpu.VMEM((_NB, _B), jnp.int32),
        pltpu.VMEM((_NB, _B), jnp.int32),
        pltpu.VMEM((_NB, _B), jnp.float32),
        pltpu.VMEM((_NBUF, _B, _D), jnp.float32),
        pltpu.VMEM_SHARED((N_NODES, _D), jnp.float32),
        pltpu.SemaphoreType.DMA,
        pltpu.SemaphoreType.DMA,
    ],
)
def _spmm(sup_hbm, src_hbm, dst_hbm, w_hbm, zero_hbm, out_hbm,
          src_v, dst_v, w_v, rows_v, acc, gsem, ssem):
    c = lax.axis_index("c")
    s = lax.axis_index("s")
    wid = s * _NC + c

    pltpu.sync_copy(src_hbm.at[wid], src_v)
    pltpu.sync_copy(dst_hbm.at[wid], dst_v)
    pltpu.sync_copy(w_hbm.at[wid], w_v)

    rbase = pl.multiple_of(s * _RPT, 8)

    @pl.when(s < _NS - 1)
    def _():
        pltpu.sync_copy(zero_hbm.at[pl.ds(rbase, _RPT)],
                        acc.at[pl.ds(rbase, _RPT)])

    @pl.when(s == _NS - 1)
    def _():
        pltpu.sync_copy(zero_hbm.at[pl.ds(rbase, _RPT_LAST)],
                        acc.at[pl.ds(rbase, _RPT_LAST)])

    plsc.subcore_barrier()

    def gather(b, u):
        return pltpu.async_copy(sup_hbm.at[src_v.at[b]], rows_v.at[u], gsem)

    for u in range(_NBUF):
        gather(u, u)

    def block(t, carry):
        for u in range(_NBUF):
            b = t * _NBUF + u
            pltpu.make_async_copy(sup_hbm.at[src_v.at[b]], rows_v.at[u],
                                  gsem).wait()

            def scale(g, cc):
                wvec = w_v[b, pl.ds(g * 16, 16)]
                for j in range(16):
                    wspl = lax.broadcast(wvec[j], (16,))
                    e = g * 16 + j
                    for k in range(_D // 16):
                        sl = pl.ds(k * 16, 16)
                        rows_v[u, e, sl] = rows_v[u, e, sl] * wspl
                return cc

            lax.fori_loop(0, _B // 16, scale, 0)
            pltpu.async_copy(rows_v.at[u], acc.at[dst_v.at[b]], ssem,
                             add=True)

        for u in range(_NBUF):
            b = t * _NBUF + u
            pltpu.make_async_copy(rows_v.at[u], acc.at[dst_v.at[b]],
                                  ssem).wait()

            @pl.when(t < _NB // _NBUF - 1)
            def _():
                gather(b + _NBUF, u)

        return carry

    lax.fori_loop(0, _NB // _NBUF, block, 0)
    plsc.subcore_barrier()

    @pl.when(s < _NS - 1)
    def _():
        pltpu.sync_copy(acc.at[pl.ds(rbase, _RPT)],
                        out_hbm.at[c, pl.ds(rbase, _RPT)])

    @pl.when(s == _NS - 1)
    def _():
        pltpu.sync_copy(acc.at[pl.ds(rbase, _RPT_LAST)],
                        out_hbm.at[c, pl.ds(rbase, _RPT_LAST)])


def _mm_split_body(x_ref, w_ref, oa_ref, ob_ref):
    sup = jnp.dot(x_ref[...], w_ref[...], preferred_element_type=jnp.float32)
    oa_ref[...] = sup[:, :_D]
    ob_ref[...] = sup[:, _D:]


def _sum_relu_mm_body(pa_ref, pb_ref, w_ref, o_ref):
    h = jnp.concatenate(
        [jnp.maximum(pa_ref[0] + pa_ref[1], 0.0),
         jnp.maximum(pb_ref[0] + pb_ref[1], 0.0)], axis=-1)
    o_ref[...] = jnp.dot(h, w_ref[...], preferred_element_type=jnp.float32)


def _sum_relu_softmax_body(p_ref, o_ref):
    z = jnp.maximum(p_ref[0] + p_ref[1], 0.0)
    z = z - jnp.max(z, axis=-1, keepdims=True)
    ez = jnp.exp(z)
    o_ref[...] = ez / jnp.sum(ez, axis=-1, keepdims=True)


def kernel(x, edge_index, edge_weight, W1, W2):
    src = edge_index[0].astype(jnp.int32)
    dst = edge_index[1].astype(jnp.int32)
    w = edge_weight.astype(jnp.float32)

    pad_per_w = (_E_PAD - N_EDGES) // _NW
    dummy_idx = (jnp.arange(_NW * pad_per_w, dtype=jnp.int32)
                 % N_NODES).reshape(_NW, pad_per_w)

    def _pad(a, filler):
        return jnp.concatenate(
            [a.reshape(_NW, N_EDGES // _NW), filler], axis=1
        ).reshape(_NW, _NB, _B)

    src_p = _pad(src, dummy_idx)
    dst_p = _pad(dst, dummy_idx)
    w_p = _pad(w, jnp.zeros((_NW, pad_per_w), jnp.float32))

    zeros64 = jnp.zeros((N_NODES, _D), jnp.float32)

    s1a, s1b = pl.pallas_call(
        _mm_split_body,
        out_shape=(jax.ShapeDtypeStruct((N_NODES, _D), jnp.float32),
                   jax.ShapeDtypeStruct((N_NODES, _D), jnp.float32)),
    )(x, W1)

    p1a = _spmm(s1a, src_p, dst_p, w_p, zeros64)
    p1b = _spmm(s1b, src_p, dst_p, w_p, zeros64)

    support2 = pl.pallas_call(
        _sum_relu_mm_body,
        out_shape=jax.ShapeDtypeStruct((N_NODES, OUT), jnp.float32),
    )(p1a, p1b, W2)

    p2 = _spmm(support2, src_p, dst_p, w_p, zeros64)

    return pl.pallas_call(
        _sum_relu_softmax_body,
        out_shape=jax.ShapeDtypeStruct((N_NODES, OUT), jnp.float32),
    )(p2)


# speedup vs baseline: 2.7179x; 1.5635x over previous
import functools

import jax
import jax.numpy as jnp
from jax import lax
from jax.experimental import pallas as pl
from jax.experimental.pallas import tpu as pltpu
from jax.experimental.pallas import tpu_sc as plsc

N_NODES = 10000
IN_F = 128
HID = 128
OUT = 64
N_EDGES = 320000

_NC = 2
_NS = 16
_NW = _NC * _NS
_B = 128
_NB = 80
_EPT = _NB * _B
_E_PAD = _NW * _EPT
_NBUF = 4
_D = 64
_RPT = 632
_RPT_LAST = N_NODES - _RPT * (_NS - 1)

_mesh = plsc.VectorSubcoreMesh(core_axis_name="c", subcore_axis_name="s")


@functools.partial(
    pl.kernel,
    mesh=_mesh,
    compiler_params=pltpu.CompilerParams(use_tc_tiling_on_sc=False),
    out_type=jax.ShapeDtypeStruct((_NC, N_NODES, _D), jnp.float32),
    scratch_types=[
        pltpu.VMEM((_NB, _B), jnp.int32),
        pltpu.VMEM((_NB, _B), jnp.int32),
        pltpu.VMEM((_NB, _B), jnp.float32),
        pltpu.VMEM((_NBUF, _B, _D), jnp.float32),
        pltpu.VMEM_SHARED((N_NODES, _D), jnp.float32),
        pltpu.SemaphoreType.DMA,
        pltpu.SemaphoreType.DMA,
    ],
)
def _spmm(sup_hbm, src_hbm, dst_hbm, w_hbm, zero_hbm, out_hbm,
          src_v, dst_v, w_v, rows_v, acc, gsem, ssem):
    c = lax.axis_index("c")
    s = lax.axis_index("s")
    wid = s * _NC + c

    pltpu.sync_copy(src_hbm.at[wid], src_v)
    pltpu.sync_copy(dst_hbm.at[wid], dst_v)
    pltpu.sync_copy(w_hbm.at[wid], w_v)

    rbase = pl.multiple_of(s * _RPT, 8)

    @pl.when(s < _NS - 1)
    def _():
        pltpu.sync_copy(zero_hbm.at[pl.ds(rbase, _RPT)],
                        acc.at[pl.ds(rbase, _RPT)])

    @pl.when(s == _NS - 1)
    def _():
        pltpu.sync_copy(zero_hbm.at[pl.ds(rbase, _RPT_LAST)],
                        acc.at[pl.ds(rbase, _RPT_LAST)])

    plsc.subcore_barrier()

    def gather(b, u):
        return pltpu.async_copy(sup_hbm.at[src_v.at[b]], rows_v.at[u], gsem)

    for u in range(_NBUF):
        gather(u, u)

    def block(t, carry):
        for u in range(_NBUF):
            b = t * _NBUF + u
            pltpu.make_async_copy(sup_hbm.at[src_v.at[b]], rows_v.at[u],
                                  gsem).wait()

            @plsc.parallel_loop(0, _B // 16, 1, unroll=2)
            def _scale(g):
                wvec = w_v[b, pl.ds(g * 16, 16)]
                for j in range(16):
                    wspl = lax.broadcast(wvec[j], (16,))
                    e = g * 16 + j
                    vals = [rows_v[u, e, pl.ds(k * 16, 16)]
                            for k in range(_D // 16)]
                    for k in range(_D // 16):
                        rows_v[u, e, pl.ds(k * 16, 16)] = vals[k] * wspl
            pltpu.async_copy(rows_v.at[u], acc.at[dst_v.at[b]], ssem,
                             add=True)

        for u in range(_NBUF):
            b = t * _NBUF + u
            pltpu.make_async_copy(rows_v.at[u], acc.at[dst_v.at[b]],
                                  ssem).wait()

            @pl.when(t < _NB // _NBUF - 1)
            def _():
                gather(b + _NBUF, u)

        return carry

    lax.fori_loop(0, _NB // _NBUF, block, 0)
    plsc.subcore_barrier()

    @pl.when(s < _NS - 1)
    def _():
        pltpu.sync_copy(acc.at[pl.ds(rbase, _RPT)],
                        out_hbm.at[c, pl.ds(rbase, _RPT)])

    @pl.when(s == _NS - 1)
    def _():
        pltpu.sync_copy(acc.at[pl.ds(rbase, _RPT_LAST)],
                        out_hbm.at[c, pl.ds(rbase, _RPT_LAST)])


def _mm_split_body(x_ref, w_ref, oa_ref, ob_ref):
    sup = jnp.dot(x_ref[...], w_ref[...], preferred_element_type=jnp.float32)
    oa_ref[...] = sup[:, :_D]
    ob_ref[...] = sup[:, _D:]


def _sum_relu_mm_body(pa_ref, pb_ref, w_ref, o_ref):
    h = jnp.concatenate(
        [jnp.maximum(pa_ref[0] + pa_ref[1], 0.0),
         jnp.maximum(pb_ref[0] + pb_ref[1], 0.0)], axis=-1)
    o_ref[...] = jnp.dot(h, w_ref[...], preferred_element_type=jnp.float32)


def _sum_relu_softmax_body(p_ref, o_ref):
    z = jnp.maximum(p_ref[0] + p_ref[1], 0.0)
    z = z - jnp.max(z, axis=-1, keepdims=True)
    ez = jnp.exp(z)
    o_ref[...] = ez / jnp.sum(ez, axis=-1, keepdims=True)


def kernel(x, edge_index, edge_weight, W1, W2):
    src = edge_index[0].astype(jnp.int32)
    dst = edge_index[1].astype(jnp.int32)
    w = edge_weight.astype(jnp.float32)

    pad_per_w = (_E_PAD - N_EDGES) // _NW
    dummy_idx = (jnp.arange(_NW * pad_per_w, dtype=jnp.int32)
                 % N_NODES).reshape(_NW, pad_per_w)

    def _pad(a, filler):
        return jnp.concatenate(
            [a.reshape(_NW, N_EDGES // _NW), filler], axis=1
        ).reshape(_NW, _NB, _B)

    src_p = _pad(src, dummy_idx)
    dst_p = _pad(dst, dummy_idx)
    w_p = _pad(w, jnp.zeros((_NW, pad_per_w), jnp.float32))

    zeros64 = jnp.zeros((N_NODES, _D), jnp.float32)

    s1a, s1b = pl.pallas_call(
        _mm_split_body,
        out_shape=(jax.ShapeDtypeStruct((N_NODES, _D), jnp.float32),
                   jax.ShapeDtypeStruct((N_NODES, _D), jnp.float32)),
    )(x, W1)

    p1a = _spmm(s1a, src_p, dst_p, w_p, zeros64)
    p1b = _spmm(s1b, src_p, dst_p, w_p, zeros64)

    support2 = pl.pallas_call(
        _sum_relu_mm_body,
        out_shape=jax.ShapeDtypeStruct((N_NODES, OUT), jnp.float32),
    )(p1a, p1b, W2)

    p2 = _spmm(support2, src_p, dst_p, w_p, zeros64)

    return pl.pallas_call(
        _sum_relu_softmax_body,
        out_shape=jax.ShapeDtypeStruct((N_NODES, OUT), jnp.float32),
    )(p2)
